# baseline (speedup 1.0000x reference)
import functools

import jax
import jax.numpy as jnp
import numpy as np
from jax import lax
from jax.experimental import pallas as pl
from jax.experimental.pallas import tpu as pltpu
from jax.experimental.pallas import tpu_sc as plsc

NUM_CLASSES = 80
NUM_ANCHORS = 20000
A_PAD = 20480
BATCH = 8
MAX_TRUE = 100
POS_THRESH = 0.5
NEG_THRESH = 0.4
EPS = 1e-7

A_BLK = 2048
NA = A_PAD // A_BLK

NWORKERS = 32
PER_W = (BATCH * A_PAD) // NWORKERS
W_PER_B = A_PAD // PER_W


def _atan_pos(x):
    big = x > 2.414213562373095
    mid = x > 0.4142135623730951
    t = jnp.where(big, -1.0 / (x + EPS),
                  jnp.where(mid, (x - 1.0) / (x + 1.0), x))
    base = jnp.where(big, np.float32(np.pi / 2),
                     jnp.where(mid, np.float32(np.pi / 4), 0.0))
    z = t * t
    p = (((8.05374449538e-2 * z - 1.38776856032e-1) * z
          + 1.99777106478e-1) * z - 3.33329491539e-1) * z * t + t
    return base + p


def _assign_body(bt_ref, conf_ref, bp_ref, anc_ref,
                 sums_ref, sel_ref, acc_ref):
    b = pl.program_id(0)
    ai = pl.program_id(1)

    @pl.when((b == 0) & (ai == 0))
    def _init():
        acc_ref[...] = jnp.zeros_like(acc_ref)

    anc = anc_ref[...]
    bt = bt_ref[0]
    ax1, ay1, ax2, ay2 = (anc[c:c + 1, :] for c in range(4))
    bx1, by1, bx2, by2 = (bt[:, c:c + 1] for c in range(4))

    ix1 = jnp.maximum(ax1, bx1)
    iy1 = jnp.maximum(ay1, by1)
    ix2 = jnp.minimum(ax2, bx2)
    iy2 = jnp.minimum(ay2, by2)
    inter = jnp.maximum(ix2 - ix1, 0.0) * jnp.maximum(iy2 - iy1, 0.0)
    area_a = jnp.maximum(ax2 - ax1, 0.0) * jnp.maximum(ay2 - ay1, 0.0)
    area_b = jnp.maximum(bx2 - bx1, 0.0) * jnp.maximum(by2 - by1, 0.0)
    iou = inter / (area_a + area_b - inter + EPS)
    valid = (bx1 > 0) | (by1 > 0) | (bx2 > 0) | (by2 > 0)
    iou = jnp.where(valid, iou, -1.0)

    max_iou = jnp.max(iou, axis=0, keepdims=True)
    lane = jax.lax.broadcasted_iota(jnp.int32, (1, A_BLK), 1)
    amask = ai * A_BLK + lane < NUM_ANCHORS
    pos = (max_iou >= POS_THRESH) & amask
    pw = pos.astype(jnp.float32)
    tw = ((max_iou >= POS_THRESH) | (max_iou < NEG_THRESH)).astype(
        jnp.float32) * amask.astype(jnp.float32)

    tidx = jax.lax.broadcasted_iota(jnp.int32, (MAX_TRUE, 1), 0)
    m = iou == max_iou
    arg = jnp.min(jnp.where(m, tidx, MAX_TRUE), axis=0, keepdims=True)
    onehot = (tidx == arg).astype(jnp.float32)

    sel_ref[0] = jnp.where(pos, arg, -1)

    p = jnp.clip(conf_ref[0], EPS, 1.0 - EPS)
    bce = -(pw * jnp.log(p) + (1.0 - pw) * jnp.log(1.0 - p))
    score_part = jnp.sum(bce * tw)

    asn = jax.lax.dot_general(
        bt, onehot, (((0,), (0,)), ((), ())),
        preferred_element_type=jnp.float32)
    x1t, y1t, x2t, y2t = (asn[c:c + 1, :] for c in range(4))
    bp = bp_ref[0]
    x1p, y1p, x2p, y2p = (bp[c:c + 1, :] for c in range(4))
    wt = jnp.maximum(x2t - x1t, 0.0)
    ht = jnp.maximum(y2t - y1t, 0.0)
    wp = jnp.maximum(x2p - x1p, 0.0)
    hp = jnp.maximum(y2p - y1p, 0.0)
    binter = jnp.maximum(jnp.minimum(x2t, x2p) - jnp.maximum(x1t, x1p), 0.0) * \
             jnp.maximum(jnp.minimum(y2t, y2p) - jnp.maximum(y1t, y1p), 0.0)
    union = wt * ht + wp * hp - binter
    biou = binter / (union + EPS)
    cw = jnp.maximum(x2t, x2p) - jnp.minimum(x1t, x1p)
    ch = jnp.maximum(y2t, y2p) - jnp.minimum(y1t, y1p)
    c2 = cw * cw + ch * ch + EPS
    rho2 = ((x1t + x2t - x1p - x2p) ** 2 + (y1t + y2t - y1p - y2p) ** 2) / 4.0
    dat = _atan_pos(wt / (ht + EPS)) - _atan_pos(wp / (hp + EPS))
    v = np.float32(4.0 / (np.pi ** 2)) * dat * dat
    alpha = v / (1.0 - biou + v + EPS)
    cl = 1.0 - (biou - rho2 / c2 - alpha * v)
    bbox_part = jnp.sum(cl * pw)

    pos_cnt = jnp.sum(pw)

    lidx = jax.lax.broadcasted_iota(jnp.int32, (1, 128), 1)
    vec = (jnp.where(lidx == 0, score_part, 0.0)
           + jnp.where(lidx == 2, bbox_part, 0.0)
           + jnp.where(lidx == 3, pos_cnt, 0.0))
    acc_ref[pl.ds(b, 1), :] += vec

    @pl.when((b == BATCH - 1) & (ai == NA - 1))
    def _fin():
        acc = acc_ref[...]
        avg = jnp.sum(jnp.maximum(acc[:, 3:4], 1.0))
        sums = jnp.sum(acc, axis=0, keepdims=True)
        sums_ref[...] = jnp.where(lidx == 3, avg,
                                  jnp.where(lidx == 1, 0.0, sums))


def _logf(x):
    xi = lax.bitcast_convert_type(x, jnp.int32)
    e = jnp.right_shift(xi, 23) - 127
    mi = jnp.bitwise_or(jnp.bitwise_and(xi, 0x7FFFFF), 0x3F800000)
    mf = lax.bitcast_convert_type(mi, jnp.float32)
    big = mf > 1.4142135
    mf = jnp.where(big, mf * 0.5, mf)
    ef0 = e.astype(jnp.float32)
    ef = jnp.where(big, ef0 + 1.0, ef0)
    r = mf - 1.0
    z = r * r
    y = r * z * ((((((((7.0376836292e-2 * r - 1.1514610310e-1) * r
        + 1.1676998740e-1) * r - 1.2420140846e-1) * r
        + 1.4249322787e-1) * r - 1.6668057665e-1) * r
        + 2.0000714765e-1) * r - 2.4999993993e-1) * r + 3.3333331174e-1)
    y = y + ef * (-2.12194440e-4)
    y = y - 0.5 * z
    return r + y + ef * 0.693359375


def _sc_focal_body(sel_hbm, logit_hbm, yt_hbm, out_hbm,
                   sel_v, aidx_v, gidx_v,
                   qa_v, qb_v, ya_v, yb_v, acc_v,
                   sem_q, sem_y):
    cid = lax.axis_index("c")
    sid = lax.axis_index("s")
    wid = sid * 2 + cid
    base = wid * PER_W
    b = wid // W_PER_B
    a0 = (wid % W_PER_B) * PER_W
    lane = lax.broadcasted_iota(jnp.int32, (16,), 0)

    pltpu.sync_copy(sel_hbm.at[pl.ds(base, PER_W)], sel_v)

    zero16i = jnp.zeros((16,), jnp.int32)
    one16i = jnp.ones((16,), jnp.int32)
    zero16 = jnp.zeros((16,), jnp.float32)

    def scan_body(i, cnt):
        v = sel_v[pl.ds(i * 16, 16)]
        m = v >= zero16i
        mi = jnp.where(m, one16i, zero16i)
        a = a0 + i * 16 + lane
        rowi = b * NUM_ANCHORS + jnp.minimum(a, NUM_ANCHORS - 1)
        yrowi = b * MAX_TRUE + jnp.maximum(v, 0)
        off = cnt + plsc.cumsum(mi) - 1
        plsc.store_scatter(aidx_v, [off], rowi * NUM_CLASSES, mask=m)
        plsc.store_scatter(gidx_v, [off], yrowi * NUM_CLASSES, mask=m)
        return cnt + jnp.sum(mi)
    cnt = lax.fori_loop(0, PER_W // 16, scan_body, jnp.int32(0))

    QROWS = NUM_ANCHORS * BATCH * NUM_CLASSES // 128
    YROWS = MAX_TRUE * BATCH * NUM_CLASSES // 128

    def chunk(j, acc):
        lm = j * 16 + lane < zero16i + cnt
        qoff = jnp.where(lm, aidx_v[pl.ds(j * 16, 16)], zero16i)
        yoff = jnp.where(lm, gidx_v[pl.ds(j * 16, 16)], zero16i)
        kq = jnp.right_shift(qoff, 7)
        sq = jnp.bitwise_and(qoff, 127)
        ky = jnp.right_shift(yoff, 7)
        sy = jnp.bitwise_and(yoff, 127)
        c1 = pltpu.async_copy(logit_hbm.at[kq], qa_v, sem_q)
        c2 = pltpu.async_copy(
            logit_hbm.at[jnp.minimum(kq + 1, QROWS - 1)], qb_v, sem_q)
        c3 = pltpu.async_copy(yt_hbm.at[ky], ya_v, sem_y)
        c4 = pltpu.async_copy(
            yt_hbm.at[jnp.minimum(ky + 1, YROWS - 1)], yb_v, sem_y)
        c1.wait()
        c2.wait()
        c3.wait()
        c4.wait()

        def col(ci, acc2):
            oq = sq + ci
            oy = sy + ci
            qv = jnp.where(
                oq < zero16i + 128,
                plsc.load_gather(qa_v, [lane, jnp.minimum(oq, 127)]),
                plsc.load_gather(qb_v, [lane, jnp.maximum(oq - 128, 0)]))
            yv = jnp.where(
                oy < zero16i + 128,
                plsc.load_gather(ya_v, [lane, jnp.minimum(oy, 127)]),
                plsc.load_gather(yb_v, [lane, jnp.maximum(oy - 128, 0)]))
            q = jnp.clip(qv, EPS, 1.0 - EPS)
            p_t = 1.0 - q + yv * (2.0 * q - 1.0)
            a_t = 0.75 - 0.5 * yv
            omp = 1.0 - p_t
            f = a_t * omp * omp * (-_logf(p_t))
            return acc2 + jnp.where(lm, f, zero16)
        return lax.fori_loop(0, NUM_CLASSES, col, acc)

    nch = (cnt + 15) // 16
    acc = lax.fori_loop(0, nch, chunk, zero16)
    acc_v[...] = acc
    pltpu.sync_copy(acc_v, out_hbm.at[wid])


def _combine_body(sums_ref, parts_ref, out_ref):
    sums = sums_ref[...]
    lidx = jax.lax.broadcasted_iota(jnp.int32, (1, 128), 1)
    avg = jnp.sum(jnp.where(lidx == 3, sums, 0.0))
    cls_sum = jnp.sum(parts_ref[...])
    losses = (jnp.where(lidx == 1, cls_sum, sums)) / avg
    bad = jnp.isnan(losses) | jnp.isinf(losses)
    out_ref[...] = jnp.where(bad, 0.0, losses)


def kernel(y_true, bbox_true, conf_pred, logit_pred, bbox_pred, anchors):
    pad = A_PAD - NUM_ANCHORS
    confT = jnp.pad(jnp.reshape(conf_pred, (BATCH, 1, NUM_ANCHORS)),
                    ((0, 0), (0, 0), (0, pad)))
    bpT = jnp.pad(jnp.transpose(bbox_pred, (0, 2, 1)),
                  ((0, 0), (0, 0), (0, pad)))
    ancT = jnp.pad(jnp.transpose(anchors, (1, 0)), ((0, 0), (0, pad)))

    sums, sel = pl.pallas_call(
        _assign_body,
        grid=(BATCH, NA),
        in_specs=[
            pl.BlockSpec((1, MAX_TRUE, 4), lambda b, ai: (b, 0, 0)),
            pl.BlockSpec((1, 1, A_BLK), lambda b, ai: (b, 0, ai)),
            pl.BlockSpec((1, 4, A_BLK), lambda b, ai: (b, 0, ai)),
            pl.BlockSpec((4, A_BLK), lambda b, ai: (0, ai)),
        ],
        out_specs=[
            pl.BlockSpec((1, 128), lambda b, ai: (0, 0)),
            pl.BlockSpec((1, 1, A_BLK), lambda b, ai: (b, 0, ai)),
        ],
        out_shape=[
            jax.ShapeDtypeStruct((1, 128), jnp.float32),
            jax.ShapeDtypeStruct((BATCH, 1, A_PAD), jnp.int32),
        ],
        scratch_shapes=[pltpu.VMEM((8, 128), jnp.float32)],
        compiler_params=pltpu.CompilerParams(
            dimension_semantics=("arbitrary", "arbitrary")),
    )(bbox_true, confT, bpT, ancT)

    sc_focal = functools.partial(
        pl.kernel,
        mesh=plsc.VectorSubcoreMesh(core_axis_name="c", subcore_axis_name="s"),
        compiler_params=pltpu.CompilerParams(
            needs_layout_passes=False, use_tc_tiling_on_sc=False),
        out_type=jax.ShapeDtypeStruct((NWORKERS, 16), jnp.float32),
        scratch_types=[
            pltpu.VMEM((PER_W,), jnp.int32),
            pltpu.VMEM((PER_W,), jnp.int32),
            pltpu.VMEM((PER_W,), jnp.int32),
            pltpu.VMEM((16, 128), jnp.float32),
            pltpu.VMEM((16, 128), jnp.float32),
            pltpu.VMEM((16, 128), jnp.float32),
            pltpu.VMEM((16, 128), jnp.float32),
            pltpu.VMEM((16,), jnp.float32),
            pltpu.SemaphoreType.DMA,
            pltpu.SemaphoreType.DMA,
        ],
    )(_sc_focal_body)
    parts = sc_focal(
        jnp.reshape(sel, (BATCH * A_PAD,)),
        jnp.reshape(logit_pred, (BATCH * NUM_ANCHORS * NUM_CLASSES // 128, 128)),
        jnp.reshape(y_true, (BATCH * MAX_TRUE * NUM_CLASSES // 128, 128)))

    out = pl.pallas_call(
        _combine_body,
        out_shape=jax.ShapeDtypeStruct((1, 128), jnp.float32),
    )(sums, parts)
    return out[0, :3]


# speedup vs baseline: 1.3612x; 1.3612x over previous
import functools

import jax
import jax.numpy as jnp
import numpy as np
from jax import lax
from jax.experimental import pallas as pl
from jax.experimental.pallas import tpu as pltpu
from jax.experimental.pallas import tpu_sc as plsc

NUM_CLASSES = 80
NUM_ANCHORS = 20000
A_PAD = 20480
BATCH = 8
MAX_TRUE = 100
POS_THRESH = 0.5
NEG_THRESH = 0.4
EPS = 1e-7

A_BLK = 2048
NA = A_PAD // A_BLK

NWORKERS = 32
PER_W = (BATCH * A_PAD) // NWORKERS
W_PER_B = A_PAD // PER_W


def _atan_pos(x):
    big = x > 2.414213562373095
    mid = x > 0.4142135623730951
    t = jnp.where(big, -1.0 / (x + EPS),
                  jnp.where(mid, (x - 1.0) / (x + 1.0), x))
    base = jnp.where(big, np.float32(np.pi / 2),
                     jnp.where(mid, np.float32(np.pi / 4), 0.0))
    z = t * t
    p = (((8.05374449538e-2 * z - 1.38776856032e-1) * z
          + 1.99777106478e-1) * z - 3.33329491539e-1) * z * t + t
    return base + p


def _assign_body(bt_ref, conf_ref, bp_ref, anc_ref,
                 sums_ref, sel_ref, acc_ref):
    b = pl.program_id(0)
    ai = pl.program_id(1)

    @pl.when((b == 0) & (ai == 0))
    def _init():
        acc_ref[...] = jnp.zeros_like(acc_ref)

    anc = anc_ref[...]
    bt = bt_ref[0]
    ax1, ay1, ax2, ay2 = (anc[c:c + 1, :] for c in range(4))
    bx1, by1, bx2, by2 = (bt[:, c:c + 1] for c in range(4))

    ix1 = jnp.maximum(ax1, bx1)
    iy1 = jnp.maximum(ay1, by1)
    ix2 = jnp.minimum(ax2, bx2)
    iy2 = jnp.minimum(ay2, by2)
    inter = jnp.maximum(ix2 - ix1, 0.0) * jnp.maximum(iy2 - iy1, 0.0)
    area_a = jnp.maximum(ax2 - ax1, 0.0) * jnp.maximum(ay2 - ay1, 0.0)
    area_b = jnp.maximum(bx2 - bx1, 0.0) * jnp.maximum(by2 - by1, 0.0)
    iou = inter / (area_a + area_b - inter + EPS)
    valid = (bx1 > 0) | (by1 > 0) | (bx2 > 0) | (by2 > 0)
    iou = jnp.where(valid, iou, -1.0)

    max_iou = jnp.max(iou, axis=0, keepdims=True)
    lane = jax.lax.broadcasted_iota(jnp.int32, (1, A_BLK), 1)
    amask = ai * A_BLK + lane < NUM_ANCHORS
    pos = (max_iou >= POS_THRESH) & amask
    pw = pos.astype(jnp.float32)
    tw = ((max_iou >= POS_THRESH) | (max_iou < NEG_THRESH)).astype(
        jnp.float32) * amask.astype(jnp.float32)

    tidx = jax.lax.broadcasted_iota(jnp.int32, (MAX_TRUE, 1), 0)
    m = iou == max_iou
    arg = jnp.min(jnp.where(m, tidx, MAX_TRUE), axis=0, keepdims=True)
    onehot = (tidx == arg).astype(jnp.float32)

    sel_ref[0] = jnp.where(pos, arg, -1)

    p = jnp.clip(conf_ref[0], EPS, 1.0 - EPS)
    bce = -(pw * jnp.log(p) + (1.0 - pw) * jnp.log(1.0 - p))
    score_part = jnp.sum(bce * tw)

    asn = jax.lax.dot_general(
        bt, onehot, (((0,), (0,)), ((), ())),
        preferred_element_type=jnp.float32)
    x1t, y1t, x2t, y2t = (asn[c:c + 1, :] for c in range(4))
    bp = bp_ref[0]
    x1p, y1p, x2p, y2p = (bp[c:c + 1, :] for c in range(4))
    wt = jnp.maximum(x2t - x1t, 0.0)
    ht = jnp.maximum(y2t - y1t, 0.0)
    wp = jnp.maximum(x2p - x1p, 0.0)
    hp = jnp.maximum(y2p - y1p, 0.0)
    binter = jnp.maximum(jnp.minimum(x2t, x2p) - jnp.maximum(x1t, x1p), 0.0) * \
             jnp.maximum(jnp.minimum(y2t, y2p) - jnp.maximum(y1t, y1p), 0.0)
    union = wt * ht + wp * hp - binter
    biou = binter / (union + EPS)
    cw = jnp.maximum(x2t, x2p) - jnp.minimum(x1t, x1p)
    ch = jnp.maximum(y2t, y2p) - jnp.minimum(y1t, y1p)
    c2 = cw * cw + ch * ch + EPS
    rho2 = ((x1t + x2t - x1p - x2p) ** 2 + (y1t + y2t - y1p - y2p) ** 2) / 4.0
    dat = _atan_pos(wt / (ht + EPS)) - _atan_pos(wp / (hp + EPS))
    v = np.float32(4.0 / (np.pi ** 2)) * dat * dat
    alpha = v / (1.0 - biou + v + EPS)
    cl = 1.0 - (biou - rho2 / c2 - alpha * v)
    bbox_part = jnp.sum(cl * pw)

    pos_cnt = jnp.sum(pw)

    lidx = jax.lax.broadcasted_iota(jnp.int32, (1, 128), 1)
    vec = (jnp.where(lidx == 0, score_part, 0.0)
           + jnp.where(lidx == 2, bbox_part, 0.0)
           + jnp.where(lidx == 3, pos_cnt, 0.0))
    acc_ref[pl.ds(b, 1), :] += vec

    @pl.when((b == BATCH - 1) & (ai == NA - 1))
    def _fin():
        acc = acc_ref[...]
        avg = jnp.sum(jnp.maximum(acc[:, 3:4], 1.0))
        sums = jnp.sum(acc, axis=0, keepdims=True)
        sums_ref[...] = jnp.where(lidx == 3, avg,
                                  jnp.where(lidx == 1, 0.0, sums))


def _logf(x):
    xi = lax.bitcast_convert_type(x, jnp.int32)
    e = jnp.right_shift(xi, 23) - 127
    mi = jnp.bitwise_or(jnp.bitwise_and(xi, 0x7FFFFF), 0x3F800000)
    mf = lax.bitcast_convert_type(mi, jnp.float32)
    big = mf > 1.4142135
    mf = jnp.where(big, mf * 0.5, mf)
    ef0 = e.astype(jnp.float32)
    ef = jnp.where(big, ef0 + 1.0, ef0)
    r = mf - 1.0
    z = r * r
    y = r * z * ((((((((7.0376836292e-2 * r - 1.1514610310e-1) * r
        + 1.1676998740e-1) * r - 1.2420140846e-1) * r
        + 1.4249322787e-1) * r - 1.6668057665e-1) * r
        + 2.0000714765e-1) * r - 2.4999993993e-1) * r + 3.3333331174e-1)
    y = y + ef * (-2.12194440e-4)
    y = y - 0.5 * z
    return r + y + ef * 0.693359375


def _sc_focal_body(sel_hbm, logit_hbm, yt_hbm, out_hbm,
                   sel_v, aidx_v, gidx_v, qrows_v, yrows_v, acc_v,
                   sem_q, sem_y):
    cid = lax.axis_index("c")
    sid = lax.axis_index("s")
    wid = sid * 2 + cid
    b = wid // W_PER_B
    a0 = (wid % W_PER_B) * PER_W
    lane = lax.broadcasted_iota(jnp.int32, (16,), 0)

    pltpu.sync_copy(sel_hbm.at[b, 0, pl.ds(a0, PER_W)], sel_v)

    zero16i = jnp.zeros((16,), jnp.int32)
    one16i = jnp.ones((16,), jnp.int32)
    zero16 = jnp.zeros((16,), jnp.float32)

    def scan_body(i, cnt):
        v = sel_v[pl.ds(i * 16, 16)]
        m = v >= zero16i
        mi = jnp.where(m, one16i, zero16i)
        a = a0 + i * 16 + lane
        off = cnt + plsc.cumsum(mi) - 1
        plsc.store_scatter(aidx_v, [off],
                           jnp.minimum(a, NUM_ANCHORS - 1), mask=m)
        plsc.store_scatter(gidx_v, [off], jnp.maximum(v, 0), mask=m)
        return cnt + jnp.sum(mi)
    cnt = lax.fori_loop(0, PER_W // 16, scan_body, jnp.int32(0))

    def chunk(j, acc):
        lm = j * 16 + lane < zero16i + cnt
        av = jnp.where(lm, aidx_v[pl.ds(j * 16, 16)], zero16i)
        gv = jnp.where(lm, gidx_v[pl.ds(j * 16, 16)], zero16i)
        copies = []
        for r in range(16):
            eq = lane == zero16i + r
            a_s = jnp.max(jnp.where(eq, av, zero16i))
            g_s = jnp.max(jnp.where(eq, gv, zero16i))
            copies.append(
                pltpu.async_copy(logit_hbm.at[b, a_s], qrows_v.at[r], sem_q))
            copies.append(
                pltpu.async_copy(yt_hbm.at[b, g_s], yrows_v.at[r], sem_y))
        for c in copies:
            c.wait()

        def col(ci, acc2):
            civ = zero16i + ci
            qv = plsc.load_gather(qrows_v, [lane, civ])
            yv = plsc.load_gather(yrows_v, [lane, civ])
            q = jnp.clip(qv, EPS, 1.0 - EPS)
            p_t = 1.0 - q + yv * (2.0 * q - 1.0)
            a_t = 0.75 - 0.5 * yv
            omp = 1.0 - p_t
            f = a_t * omp * omp * (-_logf(p_t))
            return acc2 + jnp.where(lm, f, zero16)
        return lax.fori_loop(0, NUM_CLASSES, col, acc)

    nch = (cnt + 15) // 16
    acc = lax.fori_loop(0, nch, chunk, zero16)
    acc_v[...] = acc
    pltpu.sync_copy(acc_v, out_hbm.at[wid])


def _combine_body(sums_ref, parts_ref, out_ref):
    sums = sums_ref[...]
    lidx = jax.lax.broadcasted_iota(jnp.int32, (1, 128), 1)
    avg = jnp.sum(jnp.where(lidx == 3, sums, 0.0))
    cls_sum = jnp.sum(parts_ref[...])
    losses = (jnp.where(lidx == 1, cls_sum, sums)) / avg
    bad = jnp.isnan(losses) | jnp.isinf(losses)
    out_ref[...] = jnp.where(bad, 0.0, losses)


def kernel(y_true, bbox_true, conf_pred, logit_pred, bbox_pred, anchors):
    pad = A_PAD - NUM_ANCHORS
    confT = jnp.pad(jnp.reshape(conf_pred, (BATCH, 1, NUM_ANCHORS)),
                    ((0, 0), (0, 0), (0, pad)))
    bpT = jnp.pad(jnp.transpose(bbox_pred, (0, 2, 1)),
                  ((0, 0), (0, 0), (0, pad)))
    ancT = jnp.pad(jnp.transpose(anchors, (1, 0)), ((0, 0), (0, pad)))

    sums, sel = pl.pallas_call(
        _assign_body,
        grid=(BATCH, NA),
        in_specs=[
            pl.BlockSpec((1, MAX_TRUE, 4), lambda b, ai: (b, 0, 0)),
            pl.BlockSpec((1, 1, A_BLK), lambda b, ai: (b, 0, ai)),
            pl.BlockSpec((1, 4, A_BLK), lambda b, ai: (b, 0, ai)),
            pl.BlockSpec((4, A_BLK), lambda b, ai: (0, ai)),
        ],
        out_specs=[
            pl.BlockSpec((1, 128), lambda b, ai: (0, 0)),
            pl.BlockSpec((1, 1, A_BLK), lambda b, ai: (b, 0, ai)),
        ],
        out_shape=[
            jax.ShapeDtypeStruct((1, 128), jnp.float32),
            jax.ShapeDtypeStruct((BATCH, 1, A_PAD), jnp.int32),
        ],
        scratch_shapes=[pltpu.VMEM((8, 128), jnp.float32)],
        compiler_params=pltpu.CompilerParams(
            dimension_semantics=("arbitrary", "arbitrary")),
    )(bbox_true, confT, bpT, ancT)

    sc_focal = functools.partial(
        pl.kernel,
        mesh=plsc.VectorSubcoreMesh(core_axis_name="c", subcore_axis_name="s"),
        compiler_params=pltpu.CompilerParams(needs_layout_passes=False),
        out_type=jax.ShapeDtypeStruct((NWORKERS, 16), jnp.float32),
        scratch_types=[
            pltpu.VMEM((PER_W,), jnp.int32),
            pltpu.VMEM((PER_W,), jnp.int32),
            pltpu.VMEM((PER_W,), jnp.int32),
            pltpu.VMEM((16, NUM_CLASSES), jnp.float32),
            pltpu.VMEM((16, NUM_CLASSES), jnp.float32),
            pltpu.VMEM((16,), jnp.float32),
            pltpu.SemaphoreType.DMA,
            pltpu.SemaphoreType.DMA,
        ],
    )(_sc_focal_body)
    parts = sc_focal(sel, logit_pred, y_true)

    out = pl.pallas_call(
        _combine_body,
        out_shape=jax.ShapeDtypeStruct((1, 128), jnp.float32),
    )(sums, parts)
    return out[0, :3]
